# Initial kernel scaffold; baseline (speedup 1.0000x reference)
import jax
import jax.numpy as jnp
from jax.experimental import pallas as pl


def kernel(x, edge_index, edge_attr, batch, nn1_W, nn1_b, root1, bias1, bn1_g, bn1_b, nn2_W, nn2_b, roots, biases, bns_g, bns_b, lin1_W, lin1_b, bn2_g, bn2_b, lin2_W, lin2_b):
    raise NotImplementedError("write your pallas kernel here")


# speedup vs baseline: 1.1295x; 1.1295x over previous
import functools

import jax
import jax.numpy as jnp
import numpy as np
from jax import lax
from jax.experimental import pallas as pl
from jax.experimental.pallas import tpu as pltpu
from jax.experimental.pallas import tpu_sc as plsc

_N = 10000
_E = 160000
_FEAT = 128
_EMB = 16
_NG = 64
_NC = 2
_NS = 16
_NW = _NC * _NS
_CH = 128
_CHUNKS = 40
_EPW = _CH * _CHUNKS
_EPAD = _EPW * _NW
_RPS = 632
_NP = _RPS * _NS
_B = 640

_F32 = jnp.float32
_DEF = lax.Precision.DEFAULT
_HI = lax.Precision.HIGHEST


def _dot(a, b, prec):
    return lax.dot_general(a, b, (((1,), (0,)), ((), ())),
                           precision=prec, preferred_element_type=_F32)


def _bf(v):
    return v.astype(jnp.bfloat16).astype(_F32)


_sc_mesh = plsc.VectorSubcoreMesh(core_axis_name="c", subcore_axis_name="s")


def _make_gather(width):
    @functools.partial(
        pl.kernel,
        out_type=jax.ShapeDtypeStruct((_EPAD, width), _F32),
        mesh=_sc_mesh,
        compiler_params=pltpu.CompilerParams(use_tc_tiling_on_sc=False),
        scratch_types=[
            pltpu.VMEM((_CH, width), _F32),
            pltpu.VMEM((_CH,), jnp.int32),
            pltpu.SemaphoreType.DMA,
        ],
    )
    def _gather(xtab, srcr, out, rows, sidx, sem):
        cid = lax.axis_index("c")
        sid = lax.axis_index("s")
        base = (cid * _NS + sid) * _EPW

        def _chunk(j, _):
            off = pl.multiple_of(base + j * _CH, 8)
            pltpu.sync_copy(srcr.at[pl.ds(off, _CH)], sidx)
            pltpu.async_copy(xtab.at[sidx], rows, sem).wait()
            pltpu.sync_copy(rows, out.at[pl.ds(off, _CH)])
            return 0

        lax.fori_loop(0, _CHUNKS, _chunk, 0)

    return _gather


_sc_gather_x = _make_gather(_FEAT)
_sc_gather_h = _make_gather(_EMB)


@functools.partial(
    pl.kernel,
    out_type=jax.ShapeDtypeStruct((_NC, _NP, 16), _F32),
    mesh=_sc_mesh,
    compiler_params=pltpu.CompilerParams(use_tc_tiling_on_sc=False),
    scratch_types=[
        pltpu.VMEM((_CH, 16), _F32),
        pltpu.VMEM((_CH,), jnp.int32),
        pltpu.VMEM((_RPS, 16), _F32),
        pltpu.VMEM_SHARED((_NP, 16), _F32),
    ],
)
def _sc_scatter(msgr, dstr, out, msgv, didx, zbuf, aggsh):
    cid = lax.axis_index("c")
    sid = lax.axis_index("s")
    base = (cid * _NS + sid) * _EPW

    def _zrow(i, _):
        zbuf[i, :] = jnp.zeros((16,), _F32)
        return 0
    lax.fori_loop(0, _RPS, _zrow, 0)
    pltpu.sync_copy(zbuf, aggsh.at[pl.ds(sid * _RPS, _RPS)])
    plsc.subcore_barrier()

    def _chunk(j, _):
        off = pl.multiple_of(base + j * _CH, 8)
        pltpu.sync_copy(dstr.at[pl.ds(off, _CH)], didx)
        pltpu.sync_copy(msgr.at[pl.ds(off, _CH)], msgv)
        pltpu.sync_copy(msgv, aggsh.at[didx], add=True)
        return 0

    lax.fori_loop(0, _CHUNKS, _chunk, 0)
    plsc.subcore_barrier()

    pltpu.sync_copy(aggsh.at[pl.ds(sid * _RPS, _RPS)], zbuf)
    pltpu.sync_copy(zbuf, out.at[cid, pl.ds(sid * _RPS, _RPS)])


def _msg_body(ea_ref, xg_ref, nnW_ref, nnb_ref, r_ref, s_ref, msg_ref):
    w = _dot(ea_ref[...], nnW_ref[...], _DEF) + nnb_ref[...]
    wb = _bf(w)
    xrep = _dot(xg_ref[...], r_ref[...], _DEF)
    prod = xrep * wb
    msg_ref[...] = _dot(prod, s_ref[...], _HI)


def _make_msg(in_c):
    k = in_c * 16
    grid = _EPAD // _B
    return pl.pallas_call(
        _msg_body,
        grid=(grid,),
        in_specs=[
            pl.BlockSpec((_B, 16), lambda i: (i, 0)),
            pl.BlockSpec((_B, in_c), lambda i: (i, 0)),
            pl.BlockSpec((16, k), lambda i: (0, 0)),
            pl.BlockSpec((1, k), lambda i: (0, 0)),
            pl.BlockSpec((in_c, k), lambda i: (0, 0)),
            pl.BlockSpec((k, 16), lambda i: (0, 0)),
        ],
        out_specs=pl.BlockSpec((_B, 16), lambda i: (i, 0)),
        out_shape=jax.ShapeDtypeStruct((_EPAD, 16), _F32),
    )


_msg_x = _make_msg(_FEAT)
_msg_h = _make_msg(_EMB)


def _bn_body(agg_ref, h_ref, root_ref, bias_ref, g_ref, b_ref, out_ref):
    xroot = _dot(h_ref[...], root_ref[...], _DEF)
    s = (agg_ref[0, :_N, :] + agg_ref[1, :_N, :] + xroot) + bias_ref[...]
    mu = jnp.mean(s, axis=0, keepdims=True)
    var = jnp.mean((s - mu) ** 2, axis=0, keepdims=True)
    out_ref[...] = (s - mu) * lax.rsqrt(var + 1e-5) * g_ref[...] + b_ref[...]


_tc_bn = pl.pallas_call(
    _bn_body,
    out_shape=jax.ShapeDtypeStruct((_N, 16), _F32),
)


def _last_body(agg_ref, h_ref, root_ref, bias_ref, g_ref, b_ref, batch_ref,
               l1w_ref, l1b_ref, g2_ref, b2_ref, l2w_ref, l2b_ref, out_ref):
    xroot = _dot(h_ref[...], root_ref[...], _DEF)
    s = (agg_ref[0, :_N, :] + agg_ref[1, :_N, :] + xroot) + bias_ref[...]
    mu = jnp.mean(s, axis=0, keepdims=True)
    var = jnp.mean((s - mu) ** 2, axis=0, keepdims=True)
    h = (s - mu) * lax.rsqrt(var + 1e-5) * g_ref[...] + b_ref[...]

    bcol = batch_ref[...]
    neg = jnp.float32(-jnp.inf)
    rows = []
    for g in range(_NG):
        rows.append(jnp.max(jnp.where(bcol == g, h, neg), axis=0, keepdims=True))
    pooled = jnp.concatenate(rows, axis=0)

    o = jnp.maximum(_dot(pooled, l1w_ref[...], _DEF) + l1b_ref[...], 0.0)
    mu2 = jnp.mean(o, axis=0, keepdims=True)
    var2 = jnp.mean((o - mu2) ** 2, axis=0, keepdims=True)
    o = (o - mu2) * lax.rsqrt(var2 + 1e-5) * g2_ref[...] + b2_ref[...]
    out_ref[...] = _dot(o, l2w_ref[...], _DEF) + l2b_ref[...]


_tc_last = pl.pallas_call(
    _last_body,
    out_shape=jax.ShapeDtypeStruct((_NG, 1), _F32),
)


def _expand_mats(in_c):
    k = in_c * 16
    r = np.zeros((in_c, k), np.float32)
    s = np.zeros((k, 16), np.float32)
    for i in range(in_c):
        r[i, i * 16:(i + 1) * 16] = 1.0
    for o in range(16):
        s[o::16, o] = 1.0
    return jnp.asarray(r), jnp.asarray(s)


_R_X, _S_X = _expand_mats(_FEAT)
_R_H, _S_H = _expand_mats(_EMB)


def kernel(x, edge_index, edge_attr, batch, nn1_W, nn1_b, root1, bias1,
           bn1_g, bn1_b, nn2_W, nn2_b, roots, biases, bns_g, bns_b,
           lin1_W, lin1_b, bn2_g, bn2_b, lin2_W, lin2_b):
    src = edge_index[0]
    dst = edge_index[1]
    npad = _EPAD - _E
    src_p = jnp.concatenate([src, jnp.zeros((npad,), jnp.int32)])
    dst_p = jnp.concatenate([dst, jnp.full((npad,), _N, jnp.int32)])
    ea_p = jnp.concatenate([edge_attr, jnp.zeros((npad, 16), _F32)], axis=0)

    xg = _sc_gather_x(x, src_p)
    msg = _msg_x(ea_p, xg, nn1_W, nn1_b.reshape(1, -1), _R_X, _S_X)
    aggp = _sc_scatter(msg, dst_p)
    h = _tc_bn(aggp, x, root1, bias1.reshape(1, 16),
               bn1_g.reshape(1, 16), bn1_b.reshape(1, 16))

    for i in range(2):
        hg = _sc_gather_h(h, src_p)
        msg = _msg_h(ea_p, hg, nn2_W, nn2_b.reshape(1, -1), _R_H, _S_H)
        aggp = _sc_scatter(msg, dst_p)
        if i == 0:
            h = _tc_bn(aggp, h, roots[0], biases[0].reshape(1, 16),
                       bns_g[0].reshape(1, 16), bns_b[0].reshape(1, 16))
        else:
            return _tc_last(aggp, h, roots[1], biases[1].reshape(1, 16),
                            bns_g[1].reshape(1, 16), bns_b[1].reshape(1, 16),
                            batch.reshape(_N, 1), lin1_W,
                            lin1_b.reshape(1, 128), bn2_g.reshape(1, 128),
                            bn2_b.reshape(1, 128), lin2_W,
                            lin2_b.reshape(1, 1))


# speedup vs baseline: 1.1644x; 1.0309x over previous
import functools

import jax
import jax.numpy as jnp
import numpy as np
from jax import lax
from jax.experimental import pallas as pl
from jax.experimental.pallas import tpu as pltpu
from jax.experimental.pallas import tpu_sc as plsc

_N = 10000
_E = 160000
_FEAT = 128
_EMB = 16
_NG = 64
_NC = 2
_NS = 16
_NW = _NC * _NS
_CH = 256
_CHUNKS = 20
_EPW = _CH * _CHUNKS
_EPAD = _EPW * _NW
_RPS = 632
_NP = _RPS * _NS
_B = 640

_F32 = jnp.float32
_DEF = lax.Precision.DEFAULT
_HI = lax.Precision.HIGHEST


def _dot(a, b, prec):
    return lax.dot_general(a, b, (((1,), (0,)), ((), ())),
                           precision=prec, preferred_element_type=_F32)


def _bf(v):
    return v.astype(jnp.bfloat16).astype(_F32)


_sc_mesh = plsc.VectorSubcoreMesh(core_axis_name="c", subcore_axis_name="s")


def _make_gather(width):
    @functools.partial(
        pl.kernel,
        out_type=jax.ShapeDtypeStruct((_EPAD, width), jnp.bfloat16),
        mesh=_sc_mesh,
        compiler_params=pltpu.CompilerParams(use_tc_tiling_on_sc=False),
        scratch_types=[
            pltpu.VMEM((_CH, width), jnp.bfloat16),
            pltpu.VMEM((_CH,), jnp.int32),
            pltpu.SemaphoreType.DMA,
        ],
    )
    def _gather(xtab, srcr, out, rows, sidx, sem):
        cid = lax.axis_index("c")
        sid = lax.axis_index("s")
        base = (cid * _NS + sid) * _EPW

        def _chunk(j, _):
            off = pl.multiple_of(base + j * _CH, 8)
            pltpu.sync_copy(srcr.at[pl.ds(off, _CH)], sidx)
            pltpu.async_copy(xtab.at[sidx], rows, sem).wait()
            pltpu.sync_copy(rows, out.at[pl.ds(off, _CH)])
            return 0

        lax.fori_loop(0, _CHUNKS, _chunk, 0)

    return _gather


_sc_gather_x = _make_gather(_FEAT)
_sc_gather_h = _make_gather(_EMB)


@functools.partial(
    pl.kernel,
    out_type=jax.ShapeDtypeStruct((_NC, _NP, 16), _F32),
    mesh=_sc_mesh,
    compiler_params=pltpu.CompilerParams(use_tc_tiling_on_sc=False),
    scratch_types=[
        pltpu.VMEM((_CH, 16), _F32),
        pltpu.VMEM((_CH,), jnp.int32),
        pltpu.VMEM((_RPS, 16), _F32),
        pltpu.VMEM_SHARED((_NP, 16), _F32),
    ],
)
def _sc_scatter(msgr, dstr, out, msgv, didx, zbuf, aggsh):
    cid = lax.axis_index("c")
    sid = lax.axis_index("s")
    base = (cid * _NS + sid) * _EPW

    def _zrow(i, _):
        zbuf[i, :] = jnp.zeros((16,), _F32)
        return 0
    lax.fori_loop(0, _RPS, _zrow, 0)
    pltpu.sync_copy(zbuf, aggsh.at[pl.ds(sid * _RPS, _RPS)])
    plsc.subcore_barrier()

    def _chunk(j, _):
        off = pl.multiple_of(base + j * _CH, 8)
        pltpu.sync_copy(dstr.at[pl.ds(off, _CH)], didx)
        pltpu.sync_copy(msgr.at[pl.ds(off, _CH)], msgv)
        pltpu.sync_copy(msgv, aggsh.at[didx], add=True)
        return 0

    lax.fori_loop(0, _CHUNKS, _chunk, 0)
    plsc.subcore_barrier()

    pltpu.sync_copy(aggsh.at[pl.ds(sid * _RPS, _RPS)], zbuf)
    pltpu.sync_copy(zbuf, out.at[cid, pl.ds(sid * _RPS, _RPS)])


def _msg_body(ea_ref, xg_ref, nnW_ref, nnb_ref, r_ref, s_ref, msg_ref):
    w = _dot(ea_ref[...], nnW_ref[...], _DEF) + nnb_ref[...]
    wb = _bf(w)
    xrep = _dot(xg_ref[...].astype(_F32), r_ref[...], _DEF)
    prod = xrep * wb
    msg_ref[...] = _dot(prod, s_ref[...], _HI)


def _make_msg(in_c):
    k = in_c * 16
    grid = _EPAD // _B
    return pl.pallas_call(
        _msg_body,
        grid=(grid,),
        in_specs=[
            pl.BlockSpec((_B, 16), lambda i: (i, 0)),
            pl.BlockSpec((_B, in_c), lambda i: (i, 0)),
            pl.BlockSpec((16, k), lambda i: (0, 0)),
            pl.BlockSpec((1, k), lambda i: (0, 0)),
            pl.BlockSpec((in_c, k), lambda i: (0, 0)),
            pl.BlockSpec((k, 16), lambda i: (0, 0)),
        ],
        out_specs=pl.BlockSpec((_B, 16), lambda i: (i, 0)),
        out_shape=jax.ShapeDtypeStruct((_EPAD, 16), _F32),
    )


_msg_x = _make_msg(_FEAT)
_msg_h = _make_msg(_EMB)


def _bn_body(agg_ref, h_ref, root_ref, bias_ref, g_ref, b_ref, out_ref):
    xroot = _dot(h_ref[...].astype(_F32), root_ref[...], _DEF)
    s = (agg_ref[0, :_N, :] + agg_ref[1, :_N, :] + xroot) + bias_ref[...]
    mu = jnp.mean(s, axis=0, keepdims=True)
    var = jnp.mean((s - mu) ** 2, axis=0, keepdims=True)
    h = (s - mu) * lax.rsqrt(var + 1e-5) * g_ref[...] + b_ref[...]
    out_ref[...] = h.astype(jnp.bfloat16)


_tc_bn = pl.pallas_call(
    _bn_body,
    out_shape=jax.ShapeDtypeStruct((_N, 16), jnp.bfloat16),
)


def _last_body(agg_ref, h_ref, root_ref, bias_ref, g_ref, b_ref, batch_ref,
               l1w_ref, l1b_ref, g2_ref, b2_ref, l2w_ref, l2b_ref, out_ref):
    xroot = _dot(h_ref[...].astype(_F32), root_ref[...], _DEF)
    s = (agg_ref[0, :_N, :] + agg_ref[1, :_N, :] + xroot) + bias_ref[...]
    mu = jnp.mean(s, axis=0, keepdims=True)
    var = jnp.mean((s - mu) ** 2, axis=0, keepdims=True)
    h = (s - mu) * lax.rsqrt(var + 1e-5) * g_ref[...] + b_ref[...]

    bcol = batch_ref[...]
    neg = jnp.float32(-jnp.inf)
    rows = []
    for g in range(_NG):
        rows.append(jnp.max(jnp.where(bcol == g, h, neg), axis=0, keepdims=True))
    pooled = jnp.concatenate(rows, axis=0)

    o = jnp.maximum(_dot(pooled, l1w_ref[...], _DEF) + l1b_ref[...], 0.0)
    mu2 = jnp.mean(o, axis=0, keepdims=True)
    var2 = jnp.mean((o - mu2) ** 2, axis=0, keepdims=True)
    o = (o - mu2) * lax.rsqrt(var2 + 1e-5) * g2_ref[...] + b2_ref[...]
    out_ref[...] = _dot(o, l2w_ref[...], _DEF) + l2b_ref[...]


_tc_last = pl.pallas_call(
    _last_body,
    out_shape=jax.ShapeDtypeStruct((_NG, 1), _F32),
)


def _expand_mats(in_c):
    k = in_c * 16
    r = np.zeros((in_c, k), np.float32)
    s = np.zeros((k, 16), np.float32)
    for i in range(in_c):
        r[i, i * 16:(i + 1) * 16] = 1.0
    for o in range(16):
        s[o::16, o] = 1.0
    return jnp.asarray(r), jnp.asarray(s)


_R_X, _S_X = _expand_mats(_FEAT)
_R_H, _S_H = _expand_mats(_EMB)


def kernel(x, edge_index, edge_attr, batch, nn1_W, nn1_b, root1, bias1,
           bn1_g, bn1_b, nn2_W, nn2_b, roots, biases, bns_g, bns_b,
           lin1_W, lin1_b, bn2_g, bn2_b, lin2_W, lin2_b):
    src = edge_index[0]
    dst = edge_index[1]
    npad = _EPAD - _E
    src_p = jnp.concatenate([src, jnp.zeros((npad,), jnp.int32)])
    dst_p = jnp.concatenate([dst, jnp.full((npad,), _N, jnp.int32)])
    ea_p = jnp.concatenate([edge_attr, jnp.zeros((npad, 16), _F32)], axis=0)

    xb = x.astype(jnp.bfloat16)
    xg = _sc_gather_x(xb, src_p)
    msg = _msg_x(ea_p, xg, nn1_W, nn1_b.reshape(1, -1), _R_X, _S_X)
    aggp = _sc_scatter(msg, dst_p)
    h = _tc_bn(aggp, xb, root1, bias1.reshape(1, 16),
               bn1_g.reshape(1, 16), bn1_b.reshape(1, 16))

    for i in range(2):
        hg = _sc_gather_h(h, src_p)
        msg = _msg_h(ea_p, hg, nn2_W, nn2_b.reshape(1, -1), _R_H, _S_H)
        aggp = _sc_scatter(msg, dst_p)
        if i == 0:
            h = _tc_bn(aggp, h, roots[0], biases[0].reshape(1, 16),
                       bns_g[0].reshape(1, 16), bns_b[0].reshape(1, 16))
        else:
            return _tc_last(aggp, h, roots[1], biases[1].reshape(1, 16),
                            bns_g[1].reshape(1, 16), bns_b[1].reshape(1, 16),
                            batch.reshape(_N, 1), lin1_W,
                            lin1_b.reshape(1, 128), bn2_g.reshape(1, 128),
                            bn2_b.reshape(1, 128), lin2_W,
                            lin2_b.reshape(1, 1))


# speedup vs baseline: 1.2979x; 1.1147x over previous
import functools

import jax
import jax.numpy as jnp
import numpy as np
from jax import lax
from jax.experimental import pallas as pl
from jax.experimental.pallas import tpu as pltpu
from jax.experimental.pallas import tpu_sc as plsc

_N = 10000
_E = 160000
_FEAT = 128
_EMB = 16
_NG = 64
_NC = 2
_NS = 16
_NW = _NC * _NS
_CH = 256
_CHUNKS = 20
_EPW = _CH * _CHUNKS
_EPAD = _EPW * _NW
_RPS = 632
_NP = _RPS * _NS
_B = 640

_F32 = jnp.float32
_DEF = lax.Precision.DEFAULT
_HI = lax.Precision.HIGHEST


def _dot(a, b, prec):
    return lax.dot_general(a, b, (((1,), (0,)), ((), ())),
                           precision=prec, preferred_element_type=_F32)


def _bf(v):
    return v.astype(jnp.bfloat16).astype(_F32)


_sc_mesh = plsc.VectorSubcoreMesh(core_axis_name="c", subcore_axis_name="s")


def _make_gather(width):
    @functools.partial(
        pl.kernel,
        out_type=jax.ShapeDtypeStruct((_EPAD, width), jnp.bfloat16),
        mesh=_sc_mesh,
        compiler_params=pltpu.CompilerParams(use_tc_tiling_on_sc=False),
        scratch_types=[
            pltpu.VMEM((_CH, width), jnp.bfloat16),
            pltpu.VMEM((_CH,), jnp.int32),
            pltpu.SemaphoreType.DMA,
        ],
    )
    def _gather(xtab, srcr, out, rows, sidx, sem):
        cid = lax.axis_index("c")
        sid = lax.axis_index("s")
        base = (cid * _NS + sid) * _EPW

        def _chunk(j, _):
            off = pl.multiple_of(base + j * _CH, 8)
            pltpu.sync_copy(srcr.at[pl.ds(off, _CH)], sidx)
            pltpu.async_copy(xtab.at[sidx], rows, sem).wait()
            pltpu.sync_copy(rows, out.at[pl.ds(off, _CH)])
            return 0

        lax.fori_loop(0, _CHUNKS, _chunk, 0)

    return _gather


_sc_gather_x = _make_gather(_FEAT)
_sc_gather_h = _make_gather(_EMB)


@functools.partial(
    pl.kernel,
    out_type=jax.ShapeDtypeStruct((_NC, _NP, 16), _F32),
    mesh=_sc_mesh,
    compiler_params=pltpu.CompilerParams(use_tc_tiling_on_sc=False),
    scratch_types=[
        pltpu.VMEM((_CH, 16), _F32),
        pltpu.VMEM((_CH,), jnp.int32),
        pltpu.VMEM((_RPS, 16), _F32),
        pltpu.VMEM_SHARED((_NP, 16), _F32),
    ],
)
def _sc_scatter(msgr, dstr, out, msgv, didx, zbuf, aggsh):
    cid = lax.axis_index("c")
    sid = lax.axis_index("s")
    base = (cid * _NS + sid) * _EPW

    def _zrow(i, _):
        zbuf[i, :] = jnp.zeros((16,), _F32)
        return 0
    lax.fori_loop(0, _RPS, _zrow, 0)
    pltpu.sync_copy(zbuf, aggsh.at[pl.ds(sid * _RPS, _RPS)])
    plsc.subcore_barrier()

    def _chunk(j, _):
        off = pl.multiple_of(base + j * _CH, 8)
        pltpu.sync_copy(dstr.at[pl.ds(off, _CH)], didx)
        pltpu.sync_copy(msgr.at[pl.ds(off, _CH)], msgv)
        pltpu.sync_copy(msgv, aggsh.at[didx], add=True)
        return 0

    lax.fori_loop(0, _CHUNKS, _chunk, 0)
    plsc.subcore_barrier()

    pltpu.sync_copy(aggsh.at[pl.ds(sid * _RPS, _RPS)], zbuf)
    pltpu.sync_copy(zbuf, out.at[cid, pl.ds(sid * _RPS, _RPS)])


def _msg_body(ea_ref, xg_ref, nnW_ref, nnb_ref, s_ref, msg_ref):
    w = _dot(ea_ref[...], nnW_ref[...], _DEF) + nnb_ref[...]
    wb = _bf(w)
    xgf = xg_ref[...].astype(_F32)
    xrep = jnp.concatenate([xgf] * 16, axis=1)
    prod = xrep * wb
    msg_ref[...] = _dot(prod, s_ref[...], _HI)


def _make_msg(in_c):
    k = in_c * 16
    grid = _EPAD // _B
    return pl.pallas_call(
        _msg_body,
        grid=(grid,),
        in_specs=[
            pl.BlockSpec((_B, 16), lambda i: (i, 0)),
            pl.BlockSpec((_B, in_c), lambda i: (i, 0)),
            pl.BlockSpec((16, k), lambda i: (0, 0)),
            pl.BlockSpec((1, k), lambda i: (0, 0)),
            pl.BlockSpec((k, 16), lambda i: (0, 0)),
        ],
        out_specs=pl.BlockSpec((_B, 16), lambda i: (i, 0)),
        out_shape=jax.ShapeDtypeStruct((_EPAD, 16), _F32),
    )


_msg_x = _make_msg(_FEAT)
_msg_h = _make_msg(_EMB)


def _bn_body(agg_ref, h_ref, root_ref, bias_ref, g_ref, b_ref, out_ref):
    xroot = _dot(h_ref[...].astype(_F32), root_ref[...], _DEF)
    s = (agg_ref[0, :_N, :] + agg_ref[1, :_N, :] + xroot) + bias_ref[...]
    mu = jnp.mean(s, axis=0, keepdims=True)
    var = jnp.mean((s - mu) ** 2, axis=0, keepdims=True)
    h = (s - mu) * lax.rsqrt(var + 1e-5) * g_ref[...] + b_ref[...]
    out_ref[...] = h.astype(jnp.bfloat16)


_tc_bn = pl.pallas_call(
    _bn_body,
    out_shape=jax.ShapeDtypeStruct((_N, 16), jnp.bfloat16),
)


def _last_body(agg_ref, h_ref, root_ref, bias_ref, g_ref, b_ref, batch_ref,
               e8_ref, l1w_ref, l1b_ref, g2_ref, b2_ref, l2w_ref, l2b_ref,
               out_ref):
    xroot = _dot(h_ref[...].astype(_F32), root_ref[...], _DEF)
    s = (agg_ref[0, :_N, :] + agg_ref[1, :_N, :] + xroot) + bias_ref[...]
    mu = jnp.mean(s, axis=0, keepdims=True)
    var = jnp.mean((s - mu) ** 2, axis=0, keepdims=True)
    h = (s - mu) * lax.rsqrt(var + 1e-5) * g_ref[...] + b_ref[...]

    bcol = batch_ref[...]
    neg = jnp.float32(-jnp.inf)
    ht = jnp.concatenate([h] * 8, axis=1)
    goff = lax.broadcasted_iota(jnp.int32, (1, 128), 1) // 16
    sweeps = []
    for k in range(8):
        m = bcol == (goff + 8 * k)
        sweeps.append(jnp.max(jnp.where(m, ht, neg), axis=0, keepdims=True))
    pooled8 = jnp.concatenate(sweeps, axis=0)

    tmp = _dot(e8_ref[...], pooled8, _HI)
    gmod = lax.broadcasted_iota(jnp.int32, (_NG, 1), 0) % 8
    tmp = jnp.where(gmod == goff, tmp, 0.0)
    o = jnp.maximum(_dot(tmp, l1w_ref[...], _DEF) + l1b_ref[...], 0.0)
    mu2 = jnp.mean(o, axis=0, keepdims=True)
    var2 = jnp.mean((o - mu2) ** 2, axis=0, keepdims=True)
    o = (o - mu2) * lax.rsqrt(var2 + 1e-5) * g2_ref[...] + b2_ref[...]
    out_ref[...] = _dot(o, l2w_ref[...], _DEF) + l2b_ref[...]


_tc_last = pl.pallas_call(
    _last_body,
    out_shape=jax.ShapeDtypeStruct((_NG, 1), _F32),
)


def _expand_mats(in_c):
    k = in_c * 16
    perm = np.empty((k,), np.int64)
    s = np.zeros((k, 16), np.float32)
    for o in range(16):
        for i in range(in_c):
            perm[o * in_c + i] = i * 16 + o
        s[o * in_c:(o + 1) * in_c, o] = 1.0
    return perm, jnp.asarray(s)


_P_X, _S_X = _expand_mats(_FEAT)
_P_H, _S_H = _expand_mats(_EMB)

_E8 = jnp.asarray(np.repeat(np.eye(8, dtype=np.float32), 8, axis=0))


def kernel(x, edge_index, edge_attr, batch, nn1_W, nn1_b, root1, bias1,
           bn1_g, bn1_b, nn2_W, nn2_b, roots, biases, bns_g, bns_b,
           lin1_W, lin1_b, bn2_g, bn2_b, lin2_W, lin2_b):
    src = edge_index[0]
    dst = edge_index[1]
    npad = _EPAD - _E
    src_p = jnp.concatenate([src, jnp.zeros((npad,), jnp.int32)])
    dst_p = jnp.concatenate([dst, jnp.full((npad,), _N, jnp.int32)])
    ea_p = jnp.concatenate([edge_attr, jnp.zeros((npad, 16), _F32)], axis=0)

    xb = x.astype(jnp.bfloat16)
    xg = _sc_gather_x(xb, src_p)
    msg = _msg_x(ea_p, xg, nn1_W[:, _P_X], nn1_b[_P_X].reshape(1, -1), _S_X)
    aggp = _sc_scatter(msg, dst_p)
    h = _tc_bn(aggp, xb, root1, bias1.reshape(1, 16),
               bn1_g.reshape(1, 16), bn1_b.reshape(1, 16))

    for i in range(2):
        hg = _sc_gather_h(h, src_p)
        msg = _msg_h(ea_p, hg, nn2_W[:, _P_H], nn2_b[_P_H].reshape(1, -1), _S_H)
        aggp = _sc_scatter(msg, dst_p)
        if i == 0:
            h = _tc_bn(aggp, h, roots[0], biases[0].reshape(1, 16),
                       bns_g[0].reshape(1, 16), bns_b[0].reshape(1, 16))
        else:
            l1w_tiled = jnp.tile(lin1_W, (8, 1))
            return _tc_last(aggp, h, roots[1], biases[1].reshape(1, 16),
                            bns_g[1].reshape(1, 16), bns_b[1].reshape(1, 16),
                            batch.reshape(_N, 1), _E8, l1w_tiled,
                            lin1_b.reshape(1, 128), bn2_g.reshape(1, 128),
                            bn2_b.reshape(1, 128), lin2_W,
                            lin2_b.reshape(1, 1))


# speedup vs baseline: 1.7549x; 1.3521x over previous
import functools

import jax
import jax.numpy as jnp
import numpy as np
from jax import lax
from jax.experimental import pallas as pl
from jax.experimental.pallas import tpu as pltpu
from jax.experimental.pallas import tpu_sc as plsc

_N = 10000
_E = 160000
_FEAT = 128
_EMB = 16
_NG = 64
_NC = 2
_NS = 16
_NW = _NC * _NS
_CH = 256
_CHUNKS = 20
_EPW = _CH * _CHUNKS
_EPAD = _EPW * _NW
_RPS = 632
_NP = _RPS * _NS
_B = 640

_F32 = jnp.float32
_DEF = lax.Precision.DEFAULT
_HI = lax.Precision.HIGHEST


def _dot(a, b, prec):
    return lax.dot_general(a, b, (((1,), (0,)), ((), ())),
                           precision=prec, preferred_element_type=_F32)


def _bf(v):
    return v.astype(jnp.bfloat16).astype(_F32)


_sc_mesh = plsc.VectorSubcoreMesh(core_axis_name="c", subcore_axis_name="s")


def _make_gather(width):
    @functools.partial(
        pl.kernel,
        out_type=jax.ShapeDtypeStruct((_EPAD, width), jnp.bfloat16),
        mesh=_sc_mesh,
        compiler_params=pltpu.CompilerParams(use_tc_tiling_on_sc=False),
        scratch_types=[
            pltpu.VMEM((_CH, width), jnp.bfloat16),
            pltpu.VMEM((_CH,), jnp.int32),
            pltpu.SemaphoreType.DMA,
        ],
    )
    def _gather(xtab, srcr, out, rows, sidx, sem):
        cid = lax.axis_index("c")
        sid = lax.axis_index("s")
        base = (cid * _NS + sid) * _EPW

        def _chunk(j, _):
            off = pl.multiple_of(base + j * _CH, 8)
            pltpu.sync_copy(srcr.at[pl.ds(off, _CH)], sidx)
            pltpu.async_copy(xtab.at[sidx], rows, sem).wait()
            pltpu.sync_copy(rows, out.at[pl.ds(off, _CH)])
            return 0

        lax.fori_loop(0, _CHUNKS, _chunk, 0)

    return _gather


_sc_gather_x = _make_gather(_FEAT)
_sc_gather_h = _make_gather(_EMB)


@functools.partial(
    pl.kernel,
    out_type=jax.ShapeDtypeStruct((_NC, _NP, 16), _F32),
    mesh=_sc_mesh,
    compiler_params=pltpu.CompilerParams(use_tc_tiling_on_sc=False),
    scratch_types=[
        pltpu.VMEM((_CH, 16), _F32),
        pltpu.VMEM((_CH,), jnp.int32),
        pltpu.VMEM((_RPS, 16), _F32),
        pltpu.VMEM_SHARED((_NP, 16), _F32),
    ],
)
def _sc_scatter(msgr, dstr, out, msgv, didx, zbuf, aggsh):
    cid = lax.axis_index("c")
    sid = lax.axis_index("s")
    base = (cid * _NS + sid) * _EPW

    def _zrow(i, _):
        zbuf[i, :] = jnp.zeros((16,), _F32)
        return 0
    lax.fori_loop(0, _RPS, _zrow, 0)
    pltpu.sync_copy(zbuf, aggsh.at[pl.ds(sid * _RPS, _RPS)])
    plsc.subcore_barrier()

    def _chunk(j, _):
        off = pl.multiple_of(base + j * _CH, 8)
        pltpu.sync_copy(dstr.at[pl.ds(off, _CH)], didx)
        pltpu.sync_copy(msgr.at[pl.ds(off, _CH)], msgv)
        pltpu.sync_copy(msgv, aggsh.at[didx], add=True)
        return 0

    lax.fori_loop(0, _CHUNKS, _chunk, 0)
    plsc.subcore_barrier()

    pltpu.sync_copy(aggsh.at[pl.ds(sid * _RPS, _RPS)], zbuf)
    pltpu.sync_copy(zbuf, out.at[cid, pl.ds(sid * _RPS, _RPS)])


def _msg_body(ea_ref, xg_ref, nnW_ref, nnb_ref, s_ref, msg_ref):
    w = _dot(ea_ref[...], nnW_ref[...], _DEF) + nnb_ref[...]
    wb = _bf(w)
    xgf = xg_ref[...].astype(_F32)
    xrep = jnp.concatenate([xgf] * 16, axis=1)
    prod = xrep * wb
    prod_hi = _bf(prod)
    prod_lo = prod - prod_hi
    msg_ref[...] = (_dot(prod_hi, s_ref[...], _DEF) +
                    _dot(prod_lo, s_ref[...], _DEF))


def _make_msg(in_c):
    k = in_c * 16
    grid = _EPAD // _B
    return pl.pallas_call(
        _msg_body,
        grid=(grid,),
        in_specs=[
            pl.BlockSpec((_B, 16), lambda i: (i, 0)),
            pl.BlockSpec((_B, in_c), lambda i: (i, 0)),
            pl.BlockSpec((16, k), lambda i: (0, 0)),
            pl.BlockSpec((1, k), lambda i: (0, 0)),
            pl.BlockSpec((k, 16), lambda i: (0, 0)),
        ],
        out_specs=pl.BlockSpec((_B, 16), lambda i: (i, 0)),
        out_shape=jax.ShapeDtypeStruct((_EPAD, 16), _F32),
    )


_msg_x = _make_msg(_FEAT)
_msg_h = _make_msg(_EMB)


def _bn_body(agg_ref, h_ref, root_ref, bias_ref, g_ref, b_ref, out_ref):
    xroot = _dot(h_ref[...].astype(_F32), root_ref[...], _DEF)
    s = (agg_ref[0, :_N, :] + agg_ref[1, :_N, :] + xroot) + bias_ref[...]
    mu = jnp.mean(s, axis=0, keepdims=True)
    var = jnp.mean((s - mu) ** 2, axis=0, keepdims=True)
    h = (s - mu) * lax.rsqrt(var + 1e-5) * g_ref[...] + b_ref[...]
    out_ref[...] = h.astype(jnp.bfloat16)


_tc_bn = pl.pallas_call(
    _bn_body,
    out_shape=jax.ShapeDtypeStruct((_N, 16), jnp.bfloat16),
)


def _last_body(agg_ref, h_ref, root_ref, bias_ref, g_ref, b_ref, batch_ref,
               e8_ref, l1w_ref, l1b_ref, g2_ref, b2_ref, l2w_ref, l2b_ref,
               out_ref):
    xroot = _dot(h_ref[...].astype(_F32), root_ref[...], _DEF)
    s = (agg_ref[0, :_N, :] + agg_ref[1, :_N, :] + xroot) + bias_ref[...]
    mu = jnp.mean(s, axis=0, keepdims=True)
    var = jnp.mean((s - mu) ** 2, axis=0, keepdims=True)
    h = (s - mu) * lax.rsqrt(var + 1e-5) * g_ref[...] + b_ref[...]

    bcol = batch_ref[...]
    neg = jnp.float32(-jnp.inf)
    ht = jnp.concatenate([h] * 8, axis=1)
    goff = lax.broadcasted_iota(jnp.int32, (1, 128), 1) // 16
    sweeps = []
    for k in range(8):
        m = bcol == (goff + 8 * k)
        sweeps.append(jnp.max(jnp.where(m, ht, neg), axis=0, keepdims=True))
    pooled8 = jnp.concatenate(sweeps, axis=0)

    tmp = _dot(e8_ref[...], pooled8, _HI)
    gmod = lax.broadcasted_iota(jnp.int32, (_NG, 1), 0) % 8
    tmp = jnp.where(gmod == goff, tmp, 0.0)
    o = jnp.maximum(_dot(tmp, l1w_ref[...], _DEF) + l1b_ref[...], 0.0)
    mu2 = jnp.mean(o, axis=0, keepdims=True)
    var2 = jnp.mean((o - mu2) ** 2, axis=0, keepdims=True)
    o = (o - mu2) * lax.rsqrt(var2 + 1e-5) * g2_ref[...] + b2_ref[...]
    out_ref[...] = _dot(o, l2w_ref[...], _DEF) + l2b_ref[...]


_tc_last = pl.pallas_call(
    _last_body,
    out_shape=jax.ShapeDtypeStruct((_NG, 1), _F32),
)


def _expand_mats(in_c):
    k = in_c * 16
    perm = np.empty((k,), np.int64)
    s = np.zeros((k, 16), np.float32)
    for o in range(16):
        for i in range(in_c):
            perm[o * in_c + i] = i * 16 + o
        s[o * in_c:(o + 1) * in_c, o] = 1.0
    return perm, jnp.asarray(s)


_P_X, _S_X = _expand_mats(_FEAT)
_P_H, _S_H = _expand_mats(_EMB)

_E8 = jnp.asarray(np.repeat(np.eye(8, dtype=np.float32), 8, axis=0))


def kernel(x, edge_index, edge_attr, batch, nn1_W, nn1_b, root1, bias1,
           bn1_g, bn1_b, nn2_W, nn2_b, roots, biases, bns_g, bns_b,
           lin1_W, lin1_b, bn2_g, bn2_b, lin2_W, lin2_b):
    src = edge_index[0]
    dst = edge_index[1]
    npad = _EPAD - _E
    src_p = jnp.concatenate([src, jnp.zeros((npad,), jnp.int32)])
    dst_p = jnp.concatenate([dst, jnp.full((npad,), _N, jnp.int32)])
    ea_p = jnp.concatenate([edge_attr, jnp.zeros((npad, 16), _F32)], axis=0)

    xb = x.astype(jnp.bfloat16)
    xg = _sc_gather_x(xb, src_p)
    msg = _msg_x(ea_p, xg, nn1_W[:, _P_X], nn1_b[_P_X].reshape(1, -1), _S_X)
    aggp = _sc_scatter(msg, dst_p)
    h = _tc_bn(aggp, xb, root1, bias1.reshape(1, 16),
               bn1_g.reshape(1, 16), bn1_b.reshape(1, 16))

    for i in range(2):
        hg = _sc_gather_h(h, src_p)
        msg = _msg_h(ea_p, hg, nn2_W[:, _P_H], nn2_b[_P_H].reshape(1, -1), _S_H)
        aggp = _sc_scatter(msg, dst_p)
        if i == 0:
            h = _tc_bn(aggp, h, roots[0], biases[0].reshape(1, 16),
                       bns_g[0].reshape(1, 16), bns_b[0].reshape(1, 16))
        else:
            l1w_tiled = jnp.tile(lin1_W, (8, 1))
            return _tc_last(aggp, h, roots[1], biases[1].reshape(1, 16),
                            bns_g[1].reshape(1, 16), bns_b[1].reshape(1, 16),
                            batch.reshape(_N, 1), _E8, l1w_tiled,
                            lin1_b.reshape(1, 128), bn2_g.reshape(1, 128),
                            bn2_b.reshape(1, 128), lin2_W,
                            lin2_b.reshape(1, 1))


# speedup vs baseline: 1.8127x; 1.0329x over previous
import functools

import jax
import jax.numpy as jnp
import numpy as np
from jax import lax
from jax.experimental import pallas as pl
from jax.experimental.pallas import tpu as pltpu
from jax.experimental.pallas import tpu_sc as plsc

_N = 10000
_E = 160000
_FEAT = 128
_EMB = 16
_NG = 64
_NC = 2
_NS = 16
_NW = _NC * _NS
_CH = 256
_CHUNKS = 10
_EPW = _CH * _CHUNKS
_EPAD = _EPW * _NW
_EHALF = _E // 2
_RPS = 632
_NP = _RPS * _NS
_B = 640

_F32 = jnp.float32
_DEF = lax.Precision.DEFAULT
_HI = lax.Precision.HIGHEST


def _dot(a, b, prec):
    return lax.dot_general(a, b, (((1,), (0,)), ((), ())),
                           precision=prec, preferred_element_type=_F32)


def _bf(v):
    return v.astype(jnp.bfloat16).astype(_F32)


_sc_mesh = plsc.VectorSubcoreMesh(core_axis_name="c", subcore_axis_name="s")


def _make_gather(width):
    @functools.partial(
        pl.kernel,
        out_type=jax.ShapeDtypeStruct((_EPAD, width), jnp.bfloat16),
        mesh=_sc_mesh,
        compiler_params=pltpu.CompilerParams(use_tc_tiling_on_sc=False),
        scratch_types=[
            pltpu.VMEM((_CH, width), jnp.bfloat16),
            pltpu.VMEM((_CH,), jnp.int32),
            pltpu.SemaphoreType.DMA,
        ],
    )
    def _gather(xtab, srcr, out, rows, sidx, sem):
        cid = lax.axis_index("c")
        sid = lax.axis_index("s")
        base = (cid * _NS + sid) * _EPW

        def _chunk(j, _):
            off = pl.multiple_of(base + j * _CH, 8)
            pltpu.sync_copy(srcr.at[pl.ds(off, _CH)], sidx)
            pltpu.async_copy(xtab.at[sidx], rows, sem).wait()
            pltpu.sync_copy(rows, out.at[pl.ds(off, _CH)])
            return 0

        lax.fori_loop(0, _CHUNKS, _chunk, 0)

    return _gather


_sc_gather_x = _make_gather(_FEAT)
_sc_gather_h = _make_gather(_EMB)


@functools.partial(
    pl.kernel,
    out_type=jax.ShapeDtypeStruct((_NC, _NP, 16), _F32),
    mesh=_sc_mesh,
    compiler_params=pltpu.CompilerParams(use_tc_tiling_on_sc=False),
    scratch_types=[
        pltpu.VMEM((_CH, 16), _F32),
        pltpu.VMEM((_CH,), jnp.int32),
        pltpu.VMEM((_RPS, 16), _F32),
        pltpu.VMEM_SHARED((_NP, 16), _F32),
    ],
)
def _sc_scatter(msgr, dstr, out, msgv, didx, zbuf, aggsh):
    cid = lax.axis_index("c")
    sid = lax.axis_index("s")
    base = (cid * _NS + sid) * _EPW

    def _zrow(i, _):
        zbuf[i, :] = jnp.zeros((16,), _F32)
        return 0
    lax.fori_loop(0, _RPS, _zrow, 0)
    pltpu.sync_copy(zbuf, aggsh.at[pl.ds(sid * _RPS, _RPS)])
    plsc.subcore_barrier()

    def _chunk(j, _):
        off = pl.multiple_of(base + j * _CH, 8)
        pltpu.sync_copy(dstr.at[pl.ds(off, _CH)], didx)
        pltpu.sync_copy(msgr.at[pl.ds(off, _CH)], msgv)
        pltpu.sync_copy(msgv, aggsh.at[didx], add=True)
        return 0

    lax.fori_loop(0, _CHUNKS, _chunk, 0)
    plsc.subcore_barrier()

    pltpu.sync_copy(aggsh.at[pl.ds(sid * _RPS, _RPS)], zbuf)
    pltpu.sync_copy(zbuf, out.at[cid, pl.ds(sid * _RPS, _RPS)])


def _msg_body(ea_ref, xg_ref, nnW_ref, nnb_ref, s_ref, msg_ref):
    w = _dot(ea_ref[...], nnW_ref[...], _DEF) + nnb_ref[...]
    wb = _bf(w)
    xgf = xg_ref[...].astype(_F32)
    xrep = jnp.concatenate([xgf] * 16, axis=1)
    prod = xrep * wb
    prod_hi = _bf(prod)
    prod_lo = prod - prod_hi
    msg_ref[...] = (_dot(prod_hi, s_ref[...], _DEF) +
                    _dot(prod_lo, s_ref[...], _DEF))


def _make_msg(in_c):
    k = in_c * 16
    grid = _EPAD // _B
    return pl.pallas_call(
        _msg_body,
        grid=(grid,),
        in_specs=[
            pl.BlockSpec((_B, 16), lambda i: (i, 0)),
            pl.BlockSpec((_B, in_c), lambda i: (i, 0)),
            pl.BlockSpec((16, k), lambda i: (0, 0)),
            pl.BlockSpec((1, k), lambda i: (0, 0)),
            pl.BlockSpec((k, 16), lambda i: (0, 0)),
        ],
        out_specs=pl.BlockSpec((_B, 16), lambda i: (i, 0)),
        out_shape=jax.ShapeDtypeStruct((_EPAD, 16), _F32),
    )


_msg_x = _make_msg(_FEAT)
_msg_h = _make_msg(_EMB)


def _bn_body(agga_ref, aggb_ref, h_ref, root_ref, bias_ref, g_ref, b_ref,
             out_ref):
    xroot = _dot(h_ref[...].astype(_F32), root_ref[...], _DEF)
    agg = (agga_ref[0, :_N, :] + agga_ref[1, :_N, :] +
           aggb_ref[0, :_N, :] + aggb_ref[1, :_N, :])
    s = (agg + xroot) + bias_ref[...]
    mu = jnp.mean(s, axis=0, keepdims=True)
    var = jnp.mean((s - mu) ** 2, axis=0, keepdims=True)
    h = (s - mu) * lax.rsqrt(var + 1e-5) * g_ref[...] + b_ref[...]
    out_ref[...] = h.astype(jnp.bfloat16)


_tc_bn = pl.pallas_call(
    _bn_body,
    out_shape=jax.ShapeDtypeStruct((_N, 16), jnp.bfloat16),
)


def _last_body(agga_ref, aggb_ref, h_ref, root_ref, bias_ref, g_ref, b_ref,
               batch_ref, e8_ref, l1w_ref, l1b_ref, g2_ref, b2_ref, l2w_ref,
               l2b_ref, out_ref):
    xroot = _dot(h_ref[...].astype(_F32), root_ref[...], _DEF)
    agg = (agga_ref[0, :_N, :] + agga_ref[1, :_N, :] +
           aggb_ref[0, :_N, :] + aggb_ref[1, :_N, :])
    s = (agg + xroot) + bias_ref[...]
    mu = jnp.mean(s, axis=0, keepdims=True)
    var = jnp.mean((s - mu) ** 2, axis=0, keepdims=True)
    h = (s - mu) * lax.rsqrt(var + 1e-5) * g_ref[...] + b_ref[...]

    bcol = batch_ref[...]
    neg = jnp.float32(-jnp.inf)
    ht = jnp.concatenate([h] * 8, axis=1)
    goff = lax.broadcasted_iota(jnp.int32, (1, 128), 1) // 16
    sweeps = []
    for k in range(8):
        m = bcol == (goff + 8 * k)
        sweeps.append(jnp.max(jnp.where(m, ht, neg), axis=0, keepdims=True))
    pooled8 = jnp.concatenate(sweeps, axis=0)

    tmp = _dot(e8_ref[...], pooled8, _HI)
    gmod = lax.broadcasted_iota(jnp.int32, (_NG, 1), 0) % 8
    tmp = jnp.where(gmod == goff, tmp, 0.0)
    o = jnp.maximum(_dot(tmp, l1w_ref[...], _DEF) + l1b_ref[...], 0.0)
    mu2 = jnp.mean(o, axis=0, keepdims=True)
    var2 = jnp.mean((o - mu2) ** 2, axis=0, keepdims=True)
    o = (o - mu2) * lax.rsqrt(var2 + 1e-5) * g2_ref[...] + b2_ref[...]
    out_ref[...] = _dot(o, l2w_ref[...], _DEF) + l2b_ref[...]


_tc_last = pl.pallas_call(
    _last_body,
    out_shape=jax.ShapeDtypeStruct((_NG, 1), _F32),
)


def _expand_mats(in_c):
    k = in_c * 16
    perm = np.empty((k,), np.int64)
    s = np.zeros((k, 16), np.float32)
    for o in range(16):
        for i in range(in_c):
            perm[o * in_c + i] = i * 16 + o
        s[o * in_c:(o + 1) * in_c, o] = 1.0
    return perm, jnp.asarray(s)


_P_X, _S_X = _expand_mats(_FEAT)
_P_H, _S_H = _expand_mats(_EMB)

_E8 = jnp.asarray(np.repeat(np.eye(8, dtype=np.float32), 8, axis=0))


def kernel(x, edge_index, edge_attr, batch, nn1_W, nn1_b, root1, bias1,
           bn1_g, bn1_b, nn2_W, nn2_b, roots, biases, bns_g, bns_b,
           lin1_W, lin1_b, bn2_g, bn2_b, lin2_W, lin2_b):
    src = edge_index[0]
    dst = edge_index[1]
    npad = _EPAD - _EHALF
    zpad = jnp.zeros((npad,), jnp.int32)
    dpad = jnp.full((npad,), _N, jnp.int32)
    apad = jnp.zeros((npad, 16), _F32)
    srcs = [jnp.concatenate([src[:_EHALF], zpad]),
            jnp.concatenate([src[_EHALF:], zpad])]
    dsts = [jnp.concatenate([dst[:_EHALF], dpad]),
            jnp.concatenate([dst[_EHALF:], dpad])]
    eas = [jnp.concatenate([edge_attr[:_EHALF], apad], axis=0),
           jnp.concatenate([edge_attr[_EHALF:], apad], axis=0)]

    w1p = nn1_W[:, _P_X]
    b1p = nn1_b[_P_X].reshape(1, -1)
    w2p = nn2_W[:, _P_H]
    b2p = nn2_b[_P_H].reshape(1, -1)

    xb = x.astype(jnp.bfloat16)
    aggs = []
    for k in range(2):
        xg = _sc_gather_x(xb, srcs[k])
        msg = _msg_x(eas[k], xg, w1p, b1p, _S_X)
        aggs.append(_sc_scatter(msg, dsts[k]))
    h = _tc_bn(aggs[0], aggs[1], xb, root1, bias1.reshape(1, 16),
               bn1_g.reshape(1, 16), bn1_b.reshape(1, 16))

    for i in range(2):
        aggs = []
        for k in range(2):
            hg = _sc_gather_h(h, srcs[k])
            msg = _msg_h(eas[k], hg, w2p, b2p, _S_H)
            aggs.append(_sc_scatter(msg, dsts[k]))
        if i == 0:
            h = _tc_bn(aggs[0], aggs[1], h, roots[0], biases[0].reshape(1, 16),
                       bns_g[0].reshape(1, 16), bns_b[0].reshape(1, 16))
        else:
            l1w_tiled = jnp.tile(lin1_W, (8, 1))
            return _tc_last(aggs[0], aggs[1], h, roots[1],
                            biases[1].reshape(1, 16),
                            bns_g[1].reshape(1, 16), bns_b[1].reshape(1, 16),
                            batch.reshape(_N, 1), _E8, l1w_tiled,
                            lin1_b.reshape(1, 128), bn2_g.reshape(1, 128),
                            bn2_b.reshape(1, 128), lin2_W,
                            lin2_b.reshape(1, 1))
